# baseline (speedup 1.0000x reference)
import functools

import jax
import jax.numpy as jnp
from jax import lax
from jax.experimental import pallas as pl
from jax.experimental.pallas import tpu as pltpu
from jax.experimental.pallas import tpu_sc as plsc

_NUM_CORES = 2
_NUM_SUBCORES = 16
_NUM_WORKERS = _NUM_CORES * _NUM_SUBCORES
_NBUF = 4
_LEAD = 3
_BLK = 128


def _gather_transposed(table2, idxT, batch, seq, depth):
    total = batch * seq
    n_blocks = total // _BLK
    blocks_per_worker = n_blocks // _NUM_WORKERS
    lanes_per_row = 2 * depth

    mesh = plsc.VectorSubcoreMesh(
        core_axis_name="c",
        subcore_axis_name="s",
        num_cores=_NUM_CORES,
        num_subcores=_NUM_SUBCORES,
    )

    @functools.partial(
        pl.kernel,
        out_type=jax.ShapeDtypeStruct((seq * 8, _BLK, 8 * _BLK), jnp.float32),
        mesh=mesh,
        compiler_params=pltpu.CompilerParams(
            use_tc_tiling_on_sc=False, needs_layout_passes=False
        ),
        scratch_types=[
            pltpu.VMEM((blocks_per_worker * _BLK,), jnp.int32),
            [pltpu.VMEM((_BLK, depth), jnp.float32)] * _NBUF,
            [pltpu.VMEM((8, 8 * _BLK), jnp.float32)] * _NBUF,
            [pltpu.SemaphoreType.DMA] * _NBUF,
            [pltpu.SemaphoreType.DMA] * _NBUF,
        ],
    )
    def grab(table_hbm, idx_hbm, out_hbm, idx_all, rows, tbufs, gsems, wsems):
        c = lax.axis_index("c")
        s = lax.axis_index("s")
        wid = s * _NUM_CORES + c
        blk0 = wid * blocks_per_worker
        lane_iota = lax.iota(jnp.int32, 16)

        hi = [lane_iota // 8 + 2 * d0 for d0 in range(depth // 16)]
        lo_base = (lane_iota % 8) * _BLK

        pltpu.sync_copy(
            idx_hbm.at[pl.ds(blk0 * _BLK, blocks_per_worker * _BLK)], idx_all
        )

        def gather_desc(slot, b):
            return pltpu.make_async_copy(
                table_hbm.at[idx_all.at[pl.ds(b * _BLK, _BLK)]],
                rows[slot],
                gsems[slot],
            )

        def write_desc(slot, blk):
            srow = (blk // _BLK) * 8
            bt = blk % _BLK
            return pltpu.make_async_copy(
                tbufs[slot], out_hbm.at[pl.ds(srow, 8), bt], wsems[slot]
            )

        for r in range(_LEAD):
            gather_desc(r, r).start()

        def transpose(slot):
            @plsc.parallel_loop(0, _BLK, unroll=16)
            def row(l):
                lo = lo_base + l
                for d0 in range(depth // 16):
                    v = rows[slot][l, pl.ds(d0 * 16, 16)]
                    plsc.store_scatter(tbufs[slot], [hi[d0], lo], v)

        def body(i, _):
            b0 = i * _NBUF
            for r in range(_NBUF):
                b = b0 + r
                blk = blk0 + b

                @pl.when(b >= _NBUF)
                def _():
                    write_desc(r, blk).wait()

                gather_desc(r, b).wait()
                transpose(r)
                write_desc(r, blk).start()
                nxt = b + _LEAD

                @pl.when(nxt < blocks_per_worker)
                def _():
                    gather_desc((r + _LEAD) % _NBUF, nxt).start()
            return ()

        lax.fori_loop(0, blocks_per_worker // _NBUF, body, (), unroll=False)

        for r in range(_NBUF):
            write_desc(r, blk0).wait()

    return grab(table2, idxT)


@jax.jit
def _embed(x, W_embed):
    batch, seq = x.shape
    vocab, depth = W_embed.shape
    table2 = jnp.pad(W_embed, ((0, 0), (0, 128 - depth))).reshape(2 * vocab, depth)
    idxT = (x.T.astype(jnp.int32) * 2).reshape(batch * seq)
    out5 = _gather_transposed(table2, idxT, batch, seq, depth)
    out = (
        out5.reshape(seq, 8, 128, 8, 128)
        .transpose(2, 4, 0, 1, 3)
        .reshape(batch, seq, depth)
    )
    return out


def kernel(x, W_embed):
    return _embed(x, W_embed)


# speedup vs baseline: 1.7892x; 1.7892x over previous
import functools

import jax
import jax.numpy as jnp
from jax import lax
from jax.experimental import pallas as pl
from jax.experimental.pallas import tpu as pltpu
from jax.experimental.pallas import tpu_sc as plsc

_NUM_CORES = 2
_NUM_SUBCORES = 16
_NUM_WORKERS = _NUM_CORES * _NUM_SUBCORES
_NBUF = 4
_LEAD = 3
_BLK = 128


def _gather_transposed(table2, idxT, batch, seq, depth):
    total = batch * seq
    n_blocks = total // _BLK
    blocks_per_worker = n_blocks // _NUM_WORKERS
    lanes_per_row = 2 * depth

    mesh = plsc.VectorSubcoreMesh(
        core_axis_name="c",
        subcore_axis_name="s",
        num_cores=_NUM_CORES,
        num_subcores=_NUM_SUBCORES,
    )

    @functools.partial(
        pl.kernel,
        out_type=jax.ShapeDtypeStruct((seq * 8, _BLK, 8, _BLK), jnp.float32),
        mesh=mesh,
        compiler_params=pltpu.CompilerParams(
            use_tc_tiling_on_sc=False, needs_layout_passes=False
        ),
        scratch_types=[
            pltpu.VMEM((blocks_per_worker * _BLK,), jnp.int32),
            [pltpu.VMEM((_BLK, depth), jnp.float32)] * _NBUF,
            [pltpu.VMEM((8, 8, _BLK + 1), jnp.float32)] * _NBUF,
            [pltpu.SemaphoreType.DMA] * _NBUF,
            [pltpu.SemaphoreType.DMA] * _NBUF,
        ],
    )
    def grab(table_hbm, idx_hbm, out_hbm, idx_all, rows, tbufs, gsems, wsems):
        c = lax.axis_index("c")
        s = lax.axis_index("s")
        wid = s * _NUM_CORES + c
        blk0 = wid * blocks_per_worker
        lane_iota = lax.iota(jnp.int32, 16)

        hi_dt = [lane_iota // 8 + 2 * d0 for d0 in range(depth // 16)]
        hi_ds = lane_iota % 8

        pltpu.sync_copy(
            idx_hbm.at[pl.ds(blk0 * _BLK, blocks_per_worker * _BLK)], idx_all
        )

        def gather_desc(slot, b):
            return pltpu.make_async_copy(
                table_hbm.at[idx_all.at[pl.ds(b * _BLK, _BLK)]],
                rows[slot],
                gsems[slot],
            )

        def write_desc(slot, blk):
            srow = (blk // _BLK) * 8
            bt = blk % _BLK
            return pltpu.make_async_copy(
                tbufs[slot].at[:, :, pl.ds(0, _BLK)],
                out_hbm.at[pl.ds(srow, 8), bt],
                wsems[slot],
            )

        for r in range(_LEAD):
            gather_desc(r, r).start()

        def transpose(slot):
            @plsc.parallel_loop(0, _BLK, unroll=16)
            def row(l):
                lo = jnp.full((16,), l, jnp.int32)
                for d0 in range(depth // 16):
                    v = rows[slot][l, pl.ds(d0 * 16, 16)]
                    plsc.store_scatter(tbufs[slot], [hi_dt[d0], hi_ds, lo], v)

        def body(i, _):
            b0 = i * _NBUF
            for r in range(_NBUF):
                b = b0 + r
                blk = blk0 + b

                @pl.when(b >= _NBUF)
                def _():
                    write_desc(r, blk).wait()

                gather_desc(r, b).wait()
                transpose(r)
                write_desc(r, blk).start()
                nxt = b + _LEAD

                @pl.when(nxt < blocks_per_worker)
                def _():
                    gather_desc((r + _LEAD) % _NBUF, nxt).start()
            return ()

        lax.fori_loop(0, blocks_per_worker // _NBUF, body, (), unroll=False)

        for r in range(_NBUF):
            write_desc(r, blk0).wait()

    return grab(table2, idxT)


@jax.jit
def _embed(x, W_embed):
    batch, seq = x.shape
    vocab, depth = W_embed.shape
    table2 = jnp.pad(W_embed, ((0, 0), (0, 128 - depth))).reshape(2 * vocab, depth)
    idxT = (x.T.astype(jnp.int32) * 2).reshape(batch * seq)
    out5 = _gather_transposed(table2, idxT, batch, seq, depth)
    out = (
        out5.reshape(seq, 8, 128, 8, 128)
        .transpose(2, 4, 0, 1, 3)
        .reshape(batch, seq, depth)
    )
    return out


def kernel(x, W_embed):
    return _embed(x, W_embed)
